# Initial kernel scaffold; baseline (speedup 1.0000x reference)
import jax
import jax.numpy as jnp
from jax.experimental import pallas as pl


def kernel(true_x, true_batch, pred_x, pred_batch):
    raise NotImplementedError("write your pallas kernel here")


# speedup vs baseline: 19.1693x; 19.1693x over previous
import functools

import jax
import jax.numpy as jnp
from jax import lax
from jax.experimental import pallas as pl
from jax.experimental.pallas import tpu as pltpu
from jax.experimental.pallas import tpu_sc as plsc

N = 10000
NPAD = 10240
QB = 256
TT = 512
KNN = 3
D = 128
MASKV = 1e10
TILEBIG = 1e30
INITBIG = 2e30
IBIG = 2**30


def _knn_body(c1t_ref, tb_ref, c2_ref, idx_ref, dist_ref):
    xq = c2_ref[:, 0:1]
    yq = c2_ref[:, 1:2]
    zq = c2_ref[:, 2:3]
    bq = c2_ref[:, 3:4]
    bmin = jnp.min(bq)
    bmax = jnp.max(bq)

    tb = tb_ref[...]
    t_lo = jnp.sum((tb < bmin).astype(jnp.int32))
    t_hi = jnp.sum((tb <= bmax).astype(jnp.int32))
    tile_lo = t_lo // TT
    tile_hi = (t_hi + TT - 1) // TT

    d0 = jnp.full((QB, 1), INITBIG, jnp.float32)
    d1 = jnp.full((QB, 1), INITBIG, jnp.float32)
    d2_ = jnp.full((QB, 1), INITBIG, jnp.float32)
    i0 = jnp.zeros((QB, 1), jnp.int32)
    i1 = jnp.ones((QB, 1), jnp.int32)
    i2 = jnp.full((QB, 1), 2, jnp.int32)

    def body(t, carry):
        d0, d1, d2_, i0, i1, i2 = carry
        blk = c1t_ref[pl.ds(t * 8, 8), :]
        xt = blk[0:1, :]
        yt = blk[1:2, :]
        zt = blk[2:3, :]
        bt = blk[3:4, :]
        d = (xq - xt) ** 2 + (yq - yt) ** 2 + (zq - zt) ** 2
        dm = jnp.where(bq != bt, MASKV, d)
        tidx = lax.broadcasted_iota(jnp.int32, (1, TT), 1) + t * TT
        for _ in range(KNN):
            m = jnp.min(dm, axis=1, keepdims=True)
            mi = jnp.min(jnp.where(dm == m, tidx, IBIG), axis=1, keepdims=True)
            dm = jnp.where(tidx == mi, TILEBIG, dm)
            lt0 = m < d0
            lt1 = m < d1
            lt2 = m < d2_
            d2_ = jnp.where(lt1, d1, jnp.where(lt2, m, d2_))
            i2 = jnp.where(lt1, i1, jnp.where(lt2, mi, i2))
            d1 = jnp.where(lt0, d0, jnp.where(lt1, m, d1))
            i1 = jnp.where(lt0, i0, jnp.where(lt1, mi, i1))
            d0 = jnp.where(lt0, m, d0)
            i0 = jnp.where(lt0, mi, i0)
        return d0, d1, d2_, i0, i1, i2

    d0, d1, d2_, i0, i1, i2 = lax.fori_loop(
        tile_lo, tile_hi, body, (d0, d1, d2_, i0, i1, i2))
    idx_ref[:, 0:1] = i0
    idx_ref[:, 1:2] = i1
    idx_ref[:, 2:3] = i2
    idx_ref[:, 3:8] = jnp.zeros((QB, 5), jnp.int32)
    dist_ref[:, 0:1] = d0
    dist_ref[:, 1:2] = d1
    dist_ref[:, 2:3] = d2_
    dist_ref[:, 3:8] = jnp.zeros((QB, 5), jnp.float32)


def _knn_call(c1t_grid, tb_row, c2p):
    return pl.pallas_call(
        _knn_body,
        grid=(NPAD // QB,),
        in_specs=[
            pl.BlockSpec((NPAD // TT * 8, TT), lambda i: (0, 0)),
            pl.BlockSpec((8, NPAD // 8), lambda i: (0, 0)),
            pl.BlockSpec((QB, 4), lambda i: (i, 0)),
        ],
        out_specs=[
            pl.BlockSpec((QB, 8), lambda i: (i, 0)),
            pl.BlockSpec((QB, 8), lambda i: (i, 0)),
        ],
        out_shape=[
            jax.ShapeDtypeStruct((NPAD, 8), jnp.int32),
            jax.ShapeDtypeStruct((NPAD, 8), jnp.float32),
        ],
    )(c1t_grid, tb_row, c2p)


_NC = 2
_NS = 16
_NW = _NC * _NS
ROWS = KNN * NPAD
RPW = ROWS // _NW
CH = 120


def _gather_body(tab_ref, idx_ref, out_ref, idx_v, rows_v, sem):
    wid = lax.axis_index("s") * _NC + lax.axis_index("c")
    base = wid * RPW

    def chunk(k, carry):
        off = base + k * CH
        pltpu.sync_copy(idx_ref.at[pl.ds(off, CH)], idx_v)
        pltpu.async_copy(tab_ref.at[idx_v], rows_v, sem).wait()
        pltpu.sync_copy(rows_v, out_ref.at[pl.ds(off, CH)])
        return carry

    lax.fori_loop(0, RPW // CH, chunk, 0)


@functools.cache
def _make_gather_call():
    return functools.partial(
        pl.kernel,
        mesh=plsc.VectorSubcoreMesh(core_axis_name="c", subcore_axis_name="s"),
        out_type=jax.ShapeDtypeStruct((ROWS, D), jnp.float32),
        scratch_types=[
            pltpu.VMEM((CH,), jnp.int32),
            pltpu.VMEM((CH, D), jnp.float32),
            pltpu.SemaphoreType.DMA,
        ],
    )(_gather_body)


def _gather_call(tab, idx_flat):
    return _make_gather_call()(tab, idx_flat)


def _mse_body(nb_ref, dist_ref, f2_ref, out_ref, acc_ref):
    i = pl.program_id(0)

    @pl.when(i == 0)
    def _():
        acc_ref[...] = jnp.zeros_like(acc_ref)

    num = jnp.zeros((QB, D), jnp.float32)
    den = jnp.zeros((QB, 1), jnp.float32)
    for j in range(KNN):
        w = 1.0 / jnp.clip(dist_ref[:, j:j + 1], 1e-16, None)
        num = num + nb_ref[j] * w
        den = den + w
    diff = num / den - f2_ref[...]
    qid = i * QB + lax.broadcasted_iota(jnp.int32, (QB, 1), 0)
    err = jnp.where(qid < N, diff * diff, 0.0)
    acc_ref[...] += err.reshape(QB // 8, 8, D).sum(axis=0)

    @pl.when(i == pl.num_programs(0) - 1)
    def _():
        out_ref[...] = (jnp.sum(acc_ref[...]) / jnp.float32(N * D)).reshape(1, 1)


def _mse_call(nb3, dists, f2p):
    return pl.pallas_call(
        _mse_body,
        grid=(NPAD // QB,),
        in_specs=[
            pl.BlockSpec((KNN, QB, D), lambda i: (0, i, 0)),
            pl.BlockSpec((QB, 8), lambda i: (i, 0)),
            pl.BlockSpec((QB, D), lambda i: (i, 0)),
        ],
        out_specs=pl.BlockSpec((1, 1), lambda i: (0, 0)),
        out_shape=jax.ShapeDtypeStruct((1, 1), jnp.float32),
        scratch_shapes=[pltpu.VMEM((8, D), jnp.float32)],
    )(nb3, dists, f2p)


@jax.jit
def kernel(true_x, true_batch, pred_x, pred_batch):
    tb = true_batch.astype(jnp.float32)
    pb = pred_batch.astype(jnp.float32)
    c1 = true_x[:, :3]
    c2 = pred_x[:, :3]
    f1 = true_x[:, 3:]
    f2 = pred_x[:, 3:]

    pad = NPAD - N
    tb_p = jnp.pad(tb, (0, pad), constant_values=127.0)
    pb_p = jnp.pad(pb, (0, pad), constant_values=126.0)
    c1_p = jnp.pad(c1, ((0, pad), (0, 0)), constant_values=1e8)
    c2_p = jnp.pad(c2, ((0, pad), (0, 0)))

    c1t = jnp.concatenate(
        [c1_p, tb_p[:, None], jnp.zeros((NPAD, 4), jnp.float32)], axis=1)
    c1t_grid = c1t.reshape(NPAD // TT, TT, 8).transpose(0, 2, 1).reshape(-1, TT)
    tb_row = tb_p.reshape(8, NPAD // 8)
    c2q = jnp.concatenate([c2_p, pb_p[:, None]], axis=1)

    idx8, dist8 = _knn_call(c1t_grid, tb_row, c2q)
    idx_flat = idx8[:, :KNN].T.reshape(-1)

    f1_p = jnp.pad(f1, ((0, pad), (0, 0)))
    nb = _gather_call(f1_p, idx_flat)
    nb3 = nb.reshape(KNN, NPAD, D)

    f2_p = jnp.pad(f2, ((0, pad), (0, 0)))
    out = _mse_call(nb3, dist8, f2_p)
    return out[0, 0]


# speedup vs baseline: 22.2212x; 1.1592x over previous
import functools

import jax
import jax.numpy as jnp
from jax import lax
from jax.experimental import pallas as pl
from jax.experimental.pallas import tpu as pltpu
from jax.experimental.pallas import tpu_sc as plsc

N = 10000
NPAD = 10240
QB = 256
TT = 512
KNN = 3
D = 128
MASKV = 1e10
TILEBIG = 1e30
INITBIG = 2e30
IBIG = 2**30


def _knn_body(c1t_ref, tb_ref, c2_ref, idx_ref, dist_ref):
    xq = c2_ref[:, 0:1]
    yq = c2_ref[:, 1:2]
    zq = c2_ref[:, 2:3]
    bq = c2_ref[:, 3:4]
    bmin = jnp.min(bq)
    bmax = jnp.max(bq)

    tb = tb_ref[...]
    t_lo = jnp.sum((tb < bmin).astype(jnp.int32))
    t_hi = jnp.sum((tb <= bmax).astype(jnp.int32))
    tile_lo = t_lo // TT
    tile_hi = (t_hi + TT - 1) // TT

    d0 = jnp.full((QB, 1), INITBIG, jnp.float32)
    d1 = jnp.full((QB, 1), INITBIG, jnp.float32)
    d2_ = jnp.full((QB, 1), INITBIG, jnp.float32)
    i0 = jnp.zeros((QB, 1), jnp.int32)
    i1 = jnp.ones((QB, 1), jnp.int32)
    i2 = jnp.full((QB, 1), 2, jnp.int32)

    def body(t, carry):
        d0, d1, d2_, i0, i1, i2 = carry
        blk = c1t_ref[pl.ds(t * 8, 8), :]
        xt = blk[0:1, :]
        yt = blk[1:2, :]
        zt = blk[2:3, :]
        bt = blk[3:4, :]
        d = (xq - xt) ** 2 + (yq - yt) ** 2 + (zq - zt) ** 2
        dm = jnp.where(bq != bt, MASKV, d)
        tidx = lax.broadcasted_iota(jnp.int32, (1, TT), 1) + t * TT
        for _ in range(KNN):
            m = jnp.min(dm, axis=1, keepdims=True)
            mi = jnp.min(jnp.where(dm == m, tidx, IBIG), axis=1, keepdims=True)
            dm = jnp.where(tidx == mi, TILEBIG, dm)
            lt0 = m < d0
            lt1 = m < d1
            lt2 = m < d2_
            d2_ = jnp.where(lt1, d1, jnp.where(lt2, m, d2_))
            i2 = jnp.where(lt1, i1, jnp.where(lt2, mi, i2))
            d1 = jnp.where(lt0, d0, jnp.where(lt1, m, d1))
            i1 = jnp.where(lt0, i0, jnp.where(lt1, mi, i1))
            d0 = jnp.where(lt0, m, d0)
            i0 = jnp.where(lt0, mi, i0)
        return d0, d1, d2_, i0, i1, i2

    d0, d1, d2_, i0, i1, i2 = lax.fori_loop(
        tile_lo, tile_hi, body, (d0, d1, d2_, i0, i1, i2))
    idx_ref[:, 0:1] = i0
    idx_ref[:, 1:2] = i1
    idx_ref[:, 2:3] = i2
    idx_ref[:, 3:8] = jnp.zeros((QB, 5), jnp.int32)
    dist_ref[:, 0:1] = d0
    dist_ref[:, 1:2] = d1
    dist_ref[:, 2:3] = d2_
    dist_ref[:, 3:8] = jnp.zeros((QB, 5), jnp.float32)


def _knn_call(c1t_grid, tb_row, c2p):
    return pl.pallas_call(
        _knn_body,
        grid=(NPAD // QB,),
        in_specs=[
            pl.BlockSpec((NPAD // TT * 8, TT), lambda i: (0, 0)),
            pl.BlockSpec((8, NPAD // 8), lambda i: (0, 0)),
            pl.BlockSpec((QB, 4), lambda i: (i, 0)),
        ],
        out_specs=[
            pl.BlockSpec((QB, 8), lambda i: (i, 0)),
            pl.BlockSpec((QB, 8), lambda i: (i, 0)),
        ],
        out_shape=[
            jax.ShapeDtypeStruct((NPAD, 8), jnp.int32),
            jax.ShapeDtypeStruct((NPAD, 8), jnp.float32),
        ],
    )(c1t_grid, tb_row, c2p)


_NC = 2
_NS = 16
_NW = _NC * _NS
QPW = NPAD // _NW
QCH = 40
RCH = QCH * KNN
NCHUNK = QPW // QCH
L = 16


def _lane16(ref, r, c):
    return ref[r, pl.ds(c * L, L)]


def _combine_chunk(rows_v, w_ref, f2_v, err):
    def qbody(q, err):
        w0 = 1.0 / jnp.maximum(w_ref[q * 3 + 0], 1e-16)
        w1 = 1.0 / jnp.maximum(w_ref[q * 3 + 1], 1e-16)
        w2 = 1.0 / jnp.maximum(w_ref[q * 3 + 2], 1e-16)
        inv_den = 1.0 / (w0 + w1 + w2)
        r = q * 3
        for c in range(D // L):
            num = (_lane16(rows_v, r, c) * w0
                   + _lane16(rows_v, r + 1, c) * w1
                   + _lane16(rows_v, r + 2, c) * w2)
            diff = num * inv_den - _lane16(f2_v, q, c)
            err = err + diff * diff
        return err
    return lax.fori_loop(0, QCH, qbody, err)


def _sc_body(f1_ref, idx_ref, w_ref_hbm, f2_ref, out_ref,
             idx0, idx1, rows0, rows1, f20, f21, w0_v, w1_v, err_v,
             sem0, sem1, fsem0, fsem1, wsem0, wsem1):
    cid = lax.axis_index("c")
    sid = lax.axis_index("s")
    wid = sid * _NC + cid
    qbase = wid * QPW
    rbase = qbase * KNN

    nreal = jnp.clip((N - qbase) // QCH, 0, NCHUNK)

    def start(k, idx_v, rows_v, f2_v, w_v, sem, fsem, wsem):
        off = rbase + k * RCH
        pltpu.sync_copy(idx_ref.at[pl.ds(off, RCH)], idx_v)
        pltpu.async_copy(f1_ref.at[idx_v], rows_v, sem)
        pltpu.async_copy(f2_ref.at[pl.ds(qbase + k * QCH, QCH)], f2_v, fsem)
        pltpu.async_copy(w_ref_hbm.at[pl.ds(off, RCH)], w_v, wsem)

    err = jnp.zeros((L,), jnp.float32)

    def pair_body(p, err):
        k = p * 2
        @pl.when(k + 1 < nreal)
        def _():
            start(k + 1, idx1, rows1, f21, w1_v, sem1, fsem1, wsem1)
        pltpu.make_async_copy(f1_ref.at[idx0], rows0, sem0).wait()
        pltpu.make_async_copy(f2_ref.at[pl.ds(qbase, QCH)], f20, fsem0).wait()
        pltpu.make_async_copy(w_ref_hbm.at[pl.ds(rbase, RCH)], w0_v, wsem0).wait()
        err = _combine_chunk(rows0, w0_v, f20, err)

        @pl.when(k + 2 < nreal)
        def _():
            start(k + 2, idx0, rows0, f20, w0_v, sem0, fsem0, wsem0)
        pltpu.make_async_copy(f1_ref.at[idx1], rows1, sem1).wait()
        pltpu.make_async_copy(f2_ref.at[pl.ds(qbase, QCH)], f21, fsem1).wait()
        pltpu.make_async_copy(w_ref_hbm.at[pl.ds(rbase, RCH)], w1_v, wsem1).wait()
        err = _combine_chunk(rows1, w1_v, f21, err)
        return err

    @pl.when(nreal > 0)
    def _():
        start(0, idx0, rows0, f20, w0_v, sem0, fsem0, wsem0)

    err = lax.fori_loop(0, nreal // 2, pair_body, err)
    err_v[...] = err
    pltpu.sync_copy(err_v, out_ref.at[pl.ds(wid * L, L)])


@functools.cache
def _make_sc_call():
    return functools.partial(
        pl.kernel,
        mesh=plsc.VectorSubcoreMesh(core_axis_name="c", subcore_axis_name="s"),
        out_type=jax.ShapeDtypeStruct((_NW * L,), jnp.float32),
        scratch_types=[
            pltpu.VMEM((RCH,), jnp.int32),
            pltpu.VMEM((RCH,), jnp.int32),
            pltpu.VMEM((RCH, D), jnp.float32),
            pltpu.VMEM((RCH, D), jnp.float32),
            pltpu.VMEM((QCH, D), jnp.float32),
            pltpu.VMEM((QCH, D), jnp.float32),
            pltpu.VMEM((RCH, L), jnp.float32),
            pltpu.VMEM((RCH, L), jnp.float32),
            pltpu.VMEM((L,), jnp.float32),
            pltpu.SemaphoreType.DMA,
            pltpu.SemaphoreType.DMA,
            pltpu.SemaphoreType.DMA,
            pltpu.SemaphoreType.DMA,
            pltpu.SemaphoreType.DMA,
            pltpu.SemaphoreType.DMA,
        ],
    )(_sc_body)


def _sc_call(f1, idx_flat, w16, f2):
    return _make_sc_call()(f1, idx_flat, w16, f2)


def _sum_body(p_ref, out_ref):
    out_ref[...] = (jnp.sum(p_ref[...]) / jnp.float32(N * D)).reshape(1, 1)


def _sum_call(partials):
    return pl.pallas_call(
        _sum_body,
        out_shape=jax.ShapeDtypeStruct((1, 1), jnp.float32),
    )(partials)


@jax.jit
def kernel(true_x, true_batch, pred_x, pred_batch):
    tb = true_batch.astype(jnp.float32)
    pb = pred_batch.astype(jnp.float32)
    c1 = true_x[:, :3]
    c2 = pred_x[:, :3]
    f1 = true_x[:, 3:]
    f2 = pred_x[:, 3:]

    pad = NPAD - N
    tb_p = jnp.pad(tb, (0, pad), constant_values=127.0)
    pb_p = jnp.pad(pb, (0, pad), constant_values=126.0)
    c1_p = jnp.pad(c1, ((0, pad), (0, 0)), constant_values=1e8)
    c2_p = jnp.pad(c2, ((0, pad), (0, 0)))

    c1t = jnp.concatenate(
        [c1_p, tb_p[:, None], jnp.zeros((NPAD, 4), jnp.float32)], axis=1)
    c1t_grid = c1t.reshape(NPAD // TT, TT, 8).transpose(0, 2, 1).reshape(-1, TT)
    tb_row = tb_p.reshape(8, NPAD // 8)
    c2q = jnp.concatenate([c2_p, pb_p[:, None]], axis=1)

    idx8, dist8 = _knn_call(c1t_grid, tb_row, c2q)
    idx_flat = idx8[:, :KNN].reshape(-1)
    w16 = jnp.broadcast_to(dist8[:, :KNN].reshape(-1, 1), (KNN * NPAD, 16))

    partials = _sc_call(f1, idx_flat, w16, f2)
    out = _sum_call(partials.reshape(32, 16))
    return out[0, 0]


# speedup vs baseline: 24.6093x; 1.1075x over previous
import functools

import jax
import jax.numpy as jnp
from jax import lax
from jax.experimental import pallas as pl
from jax.experimental.pallas import tpu as pltpu
from jax.experimental.pallas import tpu_sc as plsc

N = 10000
NPAD = 10240
QB = 256
TT = 512
KNN = 3
D = 128
MASKV = 1e10
TILEBIG = 1e30
INITBIG = 2e30
IBIG = 2**30


def _knn_body(c1t_ref, tb_ref, c2_ref, idx_ref, dist_ref):
    xq = c2_ref[:, 0:1]
    yq = c2_ref[:, 1:2]
    zq = c2_ref[:, 2:3]
    bq = c2_ref[:, 3:4]
    bmin = jnp.min(bq)
    bmax = jnp.max(bq)

    tb = tb_ref[...]
    t_lo = jnp.sum((tb < bmin).astype(jnp.int32))
    t_hi = jnp.sum((tb <= bmax).astype(jnp.int32))
    tile_lo = t_lo // TT
    tile_hi = (t_hi + TT - 1) // TT

    LMASK = TT - 1
    PKBIG = 0x7E000000
    INITP = 0x7F000000

    p0 = jnp.full((QB, 1), INITP, jnp.int32)
    p1 = jnp.full((QB, 1), INITP + 1, jnp.int32)
    p2 = jnp.full((QB, 1), INITP + 2, jnp.int32)
    t0 = jnp.zeros((QB, 1), jnp.int32)
    t1 = jnp.zeros((QB, 1), jnp.int32)
    t2 = jnp.zeros((QB, 1), jnp.int32)

    def body(t, carry):
        p0, p1, p2, t0, t1, t2 = carry
        blk = c1t_ref[pl.ds(t * 8, 8), :]
        xt = blk[0:1, :]
        yt = blk[1:2, :]
        zt = blk[2:3, :]
        bt = blk[3:4, :]
        d = (xq - xt) ** 2 + (yq - yt) ** 2 + (zq - zt) ** 2
        dm = jnp.where(bq != bt, MASKV, d)
        lane = lax.broadcasted_iota(jnp.int32, (1, TT), 1)
        pk = (lax.bitcast_convert_type(dm, jnp.int32) & ~LMASK) | lane
        for _ in range(KNN):
            pm = jnp.min(pk, axis=1, keepdims=True)
            pk = jnp.where(pk == pm, PKBIG, pk)
            lt0 = pm < p0
            lt1 = pm < p1
            lt2 = pm < p2
            p2 = jnp.where(lt1, p1, jnp.where(lt2, pm, p2))
            t2 = jnp.where(lt1, t1, jnp.where(lt2, t, t2))
            p1 = jnp.where(lt0, p0, jnp.where(lt1, pm, p1))
            t1 = jnp.where(lt0, t0, jnp.where(lt1, t, t1))
            p0 = jnp.where(lt0, pm, p0)
            t0 = jnp.where(lt0, t, t0)
        return p0, p1, p2, t0, t1, t2

    p0, p1, p2, t0, t1, t2 = lax.fori_loop(
        tile_lo, tile_hi, body, (p0, p1, p2, t0, t1, t2))
    idx_ref[:, 0:1] = t0 * TT + (p0 & LMASK)
    idx_ref[:, 1:2] = t1 * TT + (p1 & LMASK)
    idx_ref[:, 2:3] = t2 * TT + (p2 & LMASK)
    idx_ref[:, 3:8] = jnp.zeros((QB, 5), jnp.int32)
    dist_ref[:, 0:1] = lax.bitcast_convert_type(p0 & ~LMASK, jnp.float32)
    dist_ref[:, 1:2] = lax.bitcast_convert_type(p1 & ~LMASK, jnp.float32)
    dist_ref[:, 2:3] = lax.bitcast_convert_type(p2 & ~LMASK, jnp.float32)
    dist_ref[:, 3:8] = jnp.zeros((QB, 5), jnp.float32)


def _knn_call(c1t_grid, tb_row, c2p):
    return pl.pallas_call(
        _knn_body,
        grid=(NPAD // QB,),
        in_specs=[
            pl.BlockSpec((NPAD // TT * 8, TT), lambda i: (0, 0)),
            pl.BlockSpec((8, NPAD // 8), lambda i: (0, 0)),
            pl.BlockSpec((QB, 4), lambda i: (i, 0)),
        ],
        out_specs=[
            pl.BlockSpec((QB, 8), lambda i: (i, 0)),
            pl.BlockSpec((QB, 8), lambda i: (i, 0)),
        ],
        out_shape=[
            jax.ShapeDtypeStruct((NPAD, 8), jnp.int32),
            jax.ShapeDtypeStruct((NPAD, 8), jnp.float32),
        ],
    )(c1t_grid, tb_row, c2p)


_NC = 2
_NS = 16
_NW = _NC * _NS
QPW = NPAD // _NW
QCH = 40
RCH = QCH * KNN
NCHUNK = QPW // QCH
L = 16


def _lane16(ref, r, c):
    return ref[r, pl.ds(c * L, L)]


def _combine_chunk(rows_v, w_ref, f2_v, err):
    def qbody(q, err):
        w0 = 1.0 / jnp.maximum(w_ref[q * 3 + 0], 1e-16)
        w1 = 1.0 / jnp.maximum(w_ref[q * 3 + 1], 1e-16)
        w2 = 1.0 / jnp.maximum(w_ref[q * 3 + 2], 1e-16)
        inv_den = 1.0 / (w0 + w1 + w2)
        r = q * 3
        for c in range(D // L):
            num = (_lane16(rows_v, r, c) * w0
                   + _lane16(rows_v, r + 1, c) * w1
                   + _lane16(rows_v, r + 2, c) * w2)
            diff = num * inv_den - _lane16(f2_v, q, c)
            err = err + diff * diff
        return err
    return lax.fori_loop(0, QCH, qbody, err)


def _sc_body(f1_ref, idx_ref, w_ref_hbm, f2_ref, out_ref,
             idx0, idx1, rows0, rows1, f20, f21, w0_v, w1_v, err_v,
             sem0, sem1, fsem0, fsem1, wsem0, wsem1):
    cid = lax.axis_index("c")
    sid = lax.axis_index("s")
    wid = sid * _NC + cid
    qbase = wid * QPW
    rbase = qbase * KNN

    nreal = jnp.clip((N - qbase) // QCH, 0, NCHUNK)

    def start(k, idx_v, rows_v, f2_v, w_v, sem, fsem, wsem):
        off = rbase + k * RCH
        pltpu.sync_copy(idx_ref.at[pl.ds(off, RCH)], idx_v)
        pltpu.async_copy(f1_ref.at[idx_v], rows_v, sem)
        pltpu.async_copy(f2_ref.at[pl.ds(qbase + k * QCH, QCH)], f2_v, fsem)
        pltpu.async_copy(w_ref_hbm.at[pl.ds(off, RCH)], w_v, wsem)

    err = jnp.zeros((L,), jnp.float32)

    def pair_body(p, err):
        k = p * 2
        @pl.when(k + 1 < nreal)
        def _():
            start(k + 1, idx1, rows1, f21, w1_v, sem1, fsem1, wsem1)
        pltpu.make_async_copy(f1_ref.at[idx0], rows0, sem0).wait()
        pltpu.make_async_copy(f2_ref.at[pl.ds(qbase, QCH)], f20, fsem0).wait()
        pltpu.make_async_copy(w_ref_hbm.at[pl.ds(rbase, RCH)], w0_v, wsem0).wait()
        err = _combine_chunk(rows0, w0_v, f20, err)

        @pl.when(k + 2 < nreal)
        def _():
            start(k + 2, idx0, rows0, f20, w0_v, sem0, fsem0, wsem0)
        pltpu.make_async_copy(f1_ref.at[idx1], rows1, sem1).wait()
        pltpu.make_async_copy(f2_ref.at[pl.ds(qbase, QCH)], f21, fsem1).wait()
        pltpu.make_async_copy(w_ref_hbm.at[pl.ds(rbase, RCH)], w1_v, wsem1).wait()
        err = _combine_chunk(rows1, w1_v, f21, err)
        return err

    @pl.when(nreal > 0)
    def _():
        start(0, idx0, rows0, f20, w0_v, sem0, fsem0, wsem0)

    err = lax.fori_loop(0, nreal // 2, pair_body, err)
    err_v[...] = err
    pltpu.sync_copy(err_v, out_ref.at[pl.ds(wid * L, L)])


@functools.cache
def _make_sc_call():
    return functools.partial(
        pl.kernel,
        mesh=plsc.VectorSubcoreMesh(core_axis_name="c", subcore_axis_name="s"),
        out_type=jax.ShapeDtypeStruct((_NW * L,), jnp.float32),
        scratch_types=[
            pltpu.VMEM((RCH,), jnp.int32),
            pltpu.VMEM((RCH,), jnp.int32),
            pltpu.VMEM((RCH, D), jnp.float32),
            pltpu.VMEM((RCH, D), jnp.float32),
            pltpu.VMEM((QCH, D), jnp.float32),
            pltpu.VMEM((QCH, D), jnp.float32),
            pltpu.VMEM((RCH, L), jnp.float32),
            pltpu.VMEM((RCH, L), jnp.float32),
            pltpu.VMEM((L,), jnp.float32),
            pltpu.SemaphoreType.DMA,
            pltpu.SemaphoreType.DMA,
            pltpu.SemaphoreType.DMA,
            pltpu.SemaphoreType.DMA,
            pltpu.SemaphoreType.DMA,
            pltpu.SemaphoreType.DMA,
        ],
    )(_sc_body)


def _sc_call(f1, idx_flat, w16, f2):
    return _make_sc_call()(f1, idx_flat, w16, f2)


def _sum_body(p_ref, out_ref):
    out_ref[...] = (jnp.sum(p_ref[...]) / jnp.float32(N * D)).reshape(1, 1)


def _sum_call(partials):
    return pl.pallas_call(
        _sum_body,
        out_shape=jax.ShapeDtypeStruct((1, 1), jnp.float32),
    )(partials)


@jax.jit
def kernel(true_x, true_batch, pred_x, pred_batch):
    tb = true_batch.astype(jnp.float32)
    pb = pred_batch.astype(jnp.float32)
    c1 = true_x[:, :3]
    c2 = pred_x[:, :3]
    f1 = true_x[:, 3:]
    f2 = pred_x[:, 3:]

    pad = NPAD - N
    tb_p = jnp.pad(tb, (0, pad), constant_values=127.0)
    pb_p = jnp.pad(pb, (0, pad), constant_values=126.0)
    c1_p = jnp.pad(c1, ((0, pad), (0, 0)), constant_values=1e8)
    c2_p = jnp.pad(c2, ((0, pad), (0, 0)))

    c1t = jnp.concatenate(
        [c1_p, tb_p[:, None], jnp.zeros((NPAD, 4), jnp.float32)], axis=1)
    c1t_grid = c1t.reshape(NPAD // TT, TT, 8).transpose(0, 2, 1).reshape(-1, TT)
    tb_row = tb_p.reshape(8, NPAD // 8)
    c2q = jnp.concatenate([c2_p, pb_p[:, None]], axis=1)

    idx8, dist8 = _knn_call(c1t_grid, tb_row, c2q)
    idx_flat = idx8[:, :KNN].reshape(-1)
    w16 = jnp.broadcast_to(dist8[:, :KNN].reshape(-1, 1), (KNN * NPAD, 16))

    partials = _sc_call(f1, idx_flat, w16, f2)
    out = _sum_call(partials.reshape(32, 16))
    return out[0, 0]


# speedup vs baseline: 29.9523x; 1.2171x over previous
import functools

import jax
import jax.numpy as jnp
from jax import lax
from jax.experimental import pallas as pl
from jax.experimental.pallas import tpu as pltpu
from jax.experimental.pallas import tpu_sc as plsc

N = 10000
NPAD = 10240
QB = 256
TT = 512
KNN = 3
D = 128
MASKV = 1e10
TILEBIG = 1e30
INITBIG = 2e30
IBIG = 2**30


def _knn_body(c1t_ref, tb_ref, c2_ref, idx_ref, dist_ref):
    xq = c2_ref[:, 0:1]
    yq = c2_ref[:, 1:2]
    zq = c2_ref[:, 2:3]
    bq = c2_ref[:, 3:4]
    bmin = jnp.min(bq)
    bmax = jnp.max(bq)

    tb = tb_ref[...]
    t_lo = jnp.sum((tb < bmin).astype(jnp.int32))
    t_hi = jnp.sum((tb <= bmax).astype(jnp.int32))
    tile_lo = t_lo // TT
    tile_hi = (t_hi + TT - 1) // TT

    IMASK = (1 << 14) - 1
    INITP = 0x7F000000
    INFP = 0x7F800000

    cq = c2_ref[:, 0:3]
    qsq = jnp.sum(cq * cq, axis=1, keepdims=True)

    s0 = jnp.full((QB, TT // 4), INITP, jnp.int32)
    s1 = jnp.full((QB, TT // 4), INITP + 1, jnp.int32)
    s2 = jnp.full((QB, TT // 4), INITP + 2, jnp.int32)

    def body(t, carry):
        s0, s1, s2 = carry
        blk = c1t_ref[pl.ds(t * 8, 8), :]
        ct = blk[0:3, :]
        bt = blk[3:4, :]
        tsq = jnp.sum(ct * ct, axis=0, keepdims=True)
        dot = lax.dot_general(cq, ct, (((1,), (0,)), ((), ())),
                              preferred_element_type=jnp.float32)
        d = (qsq + tsq) - 2.0 * dot
        dm = jnp.where(bq != bt, MASKV, jnp.maximum(d, 0.0))
        gidx = lax.broadcasted_iota(jnp.int32, (1, TT), 1) + t * TT
        pk = (lax.bitcast_convert_type(dm, jnp.int32) & ~IMASK) | gidx
        for c in range(4):
            q = pk[:, c * (TT // 4):(c + 1) * (TT // 4)]
            lt0 = q < s0
            lt1 = q < s1
            lt2 = q < s2
            s2 = jnp.where(lt1, s1, jnp.where(lt2, q, s2))
            s1 = jnp.where(lt0, s0, jnp.where(lt1, q, s1))
            s0 = jnp.where(lt0, q, s0)
        return s0, s1, s2

    s0, s1, s2 = lax.fori_loop(tile_lo, tile_hi, body, (s0, s1, s2))

    es = []
    for _ in range(KNN):
        e = jnp.min(s0, axis=1, keepdims=True)
        es.append(e)
        hit = s0 == e
        s0 = jnp.where(hit, s1, s0)
        s1 = jnp.where(hit, s2, s1)
        s2 = jnp.where(hit, INFP, s2)
    e0, e1, e2 = es
    idx_ref[:, 0:1] = e0 & IMASK
    idx_ref[:, 1:2] = e1 & IMASK
    idx_ref[:, 2:3] = e2 & IMASK
    idx_ref[:, 3:8] = jnp.zeros((QB, 5), jnp.int32)
    dist_ref[:, 0:1] = lax.bitcast_convert_type(e0 & ~IMASK, jnp.float32)
    dist_ref[:, 1:2] = lax.bitcast_convert_type(e1 & ~IMASK, jnp.float32)
    dist_ref[:, 2:3] = lax.bitcast_convert_type(e2 & ~IMASK, jnp.float32)
    dist_ref[:, 3:8] = jnp.zeros((QB, 5), jnp.float32)


def _knn_call(c1t_grid, tb_row, c2p):
    return pl.pallas_call(
        _knn_body,
        grid=(NPAD // QB,),
        in_specs=[
            pl.BlockSpec((NPAD // TT * 8, TT), lambda i: (0, 0)),
            pl.BlockSpec((8, NPAD // 8), lambda i: (0, 0)),
            pl.BlockSpec((QB, 4), lambda i: (i, 0)),
        ],
        out_specs=[
            pl.BlockSpec((QB, 8), lambda i: (i, 0)),
            pl.BlockSpec((QB, 8), lambda i: (i, 0)),
        ],
        out_shape=[
            jax.ShapeDtypeStruct((NPAD, 8), jnp.int32),
            jax.ShapeDtypeStruct((NPAD, 8), jnp.float32),
        ],
    )(c1t_grid, tb_row, c2p)


_NC = 2
_NS = 16
_NW = _NC * _NS
QPW = NPAD // _NW
QCH = 40
RCH = QCH * KNN
NCHUNK = QPW // QCH
L = 16


def _lane16(ref, r, c):
    return ref[r, pl.ds(c * L, L)]


def _combine_chunk(rows_v, w_ref, f2_v, err):
    def qbody(q, err):
        w0 = 1.0 / jnp.maximum(w_ref[q * 3 + 0], 1e-16)
        w1 = 1.0 / jnp.maximum(w_ref[q * 3 + 1], 1e-16)
        w2 = 1.0 / jnp.maximum(w_ref[q * 3 + 2], 1e-16)
        inv_den = 1.0 / (w0 + w1 + w2)
        r = q * 3
        for c in range(D // L):
            num = (_lane16(rows_v, r, c) * w0
                   + _lane16(rows_v, r + 1, c) * w1
                   + _lane16(rows_v, r + 2, c) * w2)
            diff = num * inv_den - _lane16(f2_v, q, c)
            err = err + diff * diff
        return err
    return lax.fori_loop(0, QCH, qbody, err)


def _sc_body(f1_ref, idx_ref, w_ref_hbm, f2_ref, out_ref,
             idx0, idx1, rows0, rows1, f20, f21, w0_v, w1_v, err_v,
             sem0, sem1, fsem0, fsem1, wsem0, wsem1):
    cid = lax.axis_index("c")
    sid = lax.axis_index("s")
    wid = sid * _NC + cid
    qbase = wid * QPW
    rbase = qbase * KNN

    nreal = jnp.clip((N - qbase) // QCH, 0, NCHUNK)

    def start(k, idx_v, rows_v, f2_v, w_v, sem, fsem, wsem):
        off = rbase + k * RCH
        pltpu.sync_copy(idx_ref.at[pl.ds(off, RCH)], idx_v)
        pltpu.async_copy(f1_ref.at[idx_v], rows_v, sem)
        pltpu.async_copy(f2_ref.at[pl.ds(qbase + k * QCH, QCH)], f2_v, fsem)
        pltpu.async_copy(w_ref_hbm.at[pl.ds(off, RCH)], w_v, wsem)

    err = jnp.zeros((L,), jnp.float32)

    def pair_body(p, err):
        k = p * 2
        @pl.when(k + 1 < nreal)
        def _():
            start(k + 1, idx1, rows1, f21, w1_v, sem1, fsem1, wsem1)
        pltpu.make_async_copy(f1_ref.at[idx0], rows0, sem0).wait()
        pltpu.make_async_copy(f2_ref.at[pl.ds(qbase, QCH)], f20, fsem0).wait()
        pltpu.make_async_copy(w_ref_hbm.at[pl.ds(rbase, RCH)], w0_v, wsem0).wait()
        err = _combine_chunk(rows0, w0_v, f20, err)

        @pl.when(k + 2 < nreal)
        def _():
            start(k + 2, idx0, rows0, f20, w0_v, sem0, fsem0, wsem0)
        pltpu.make_async_copy(f1_ref.at[idx1], rows1, sem1).wait()
        pltpu.make_async_copy(f2_ref.at[pl.ds(qbase, QCH)], f21, fsem1).wait()
        pltpu.make_async_copy(w_ref_hbm.at[pl.ds(rbase, RCH)], w1_v, wsem1).wait()
        err = _combine_chunk(rows1, w1_v, f21, err)
        return err

    @pl.when(nreal > 0)
    def _():
        start(0, idx0, rows0, f20, w0_v, sem0, fsem0, wsem0)

    err = lax.fori_loop(0, nreal // 2, pair_body, err)
    err_v[...] = err
    pltpu.sync_copy(err_v, out_ref.at[pl.ds(wid * L, L)])


@functools.cache
def _make_sc_call():
    return functools.partial(
        pl.kernel,
        mesh=plsc.VectorSubcoreMesh(core_axis_name="c", subcore_axis_name="s"),
        out_type=jax.ShapeDtypeStruct((_NW * L,), jnp.float32),
        scratch_types=[
            pltpu.VMEM((RCH,), jnp.int32),
            pltpu.VMEM((RCH,), jnp.int32),
            pltpu.VMEM((RCH, D), jnp.float32),
            pltpu.VMEM((RCH, D), jnp.float32),
            pltpu.VMEM((QCH, D), jnp.float32),
            pltpu.VMEM((QCH, D), jnp.float32),
            pltpu.VMEM((RCH, L), jnp.float32),
            pltpu.VMEM((RCH, L), jnp.float32),
            pltpu.VMEM((L,), jnp.float32),
            pltpu.SemaphoreType.DMA,
            pltpu.SemaphoreType.DMA,
            pltpu.SemaphoreType.DMA,
            pltpu.SemaphoreType.DMA,
            pltpu.SemaphoreType.DMA,
            pltpu.SemaphoreType.DMA,
        ],
    )(_sc_body)


def _sc_call(f1, idx_flat, w16, f2):
    return _make_sc_call()(f1, idx_flat, w16, f2)


def _sum_body(p_ref, out_ref):
    out_ref[...] = (jnp.sum(p_ref[...]) / jnp.float32(N * D)).reshape(1, 1)


def _sum_call(partials):
    return pl.pallas_call(
        _sum_body,
        out_shape=jax.ShapeDtypeStruct((1, 1), jnp.float32),
    )(partials)


@jax.jit
def kernel(true_x, true_batch, pred_x, pred_batch):
    tb = true_batch.astype(jnp.float32)
    pb = pred_batch.astype(jnp.float32)
    c1 = true_x[:, :3]
    c2 = pred_x[:, :3]
    f1 = true_x[:, 3:]
    f2 = pred_x[:, 3:]

    pad = NPAD - N
    tb_p = jnp.pad(tb, (0, pad), constant_values=127.0)
    pb_p = jnp.pad(pb, (0, pad), constant_values=126.0)
    c1_p = jnp.pad(c1, ((0, pad), (0, 0)), constant_values=1e8)
    c2_p = jnp.pad(c2, ((0, pad), (0, 0)))

    c1t = jnp.concatenate(
        [c1_p, tb_p[:, None], jnp.zeros((NPAD, 4), jnp.float32)], axis=1)
    c1t_grid = c1t.reshape(NPAD // TT, TT, 8).transpose(0, 2, 1).reshape(-1, TT)
    tb_row = tb_p.reshape(8, NPAD // 8)
    c2q = jnp.concatenate([c2_p, pb_p[:, None]], axis=1)

    idx8, dist8 = _knn_call(c1t_grid, tb_row, c2q)
    idx_flat = idx8[:, :KNN].reshape(-1)
    w16 = jnp.broadcast_to(dist8[:, :KNN].reshape(-1, 1), (KNN * NPAD, 16))

    partials = _sc_call(f1, idx_flat, w16, f2)
    out = _sum_call(partials.reshape(32, 16))
    return out[0, 0]
